# baseline (speedup 1.0000x reference)
import jax
import jax.numpy as jnp
from jax.experimental import pallas as pl

_BL = 2048


def _reduce_body(eo_ref, el_ref, ss_ref, ml_ref):
    x = eo_ref[...]
    s = x * x
    h = s[:, :128] + s[:, 128:]
    w = 64
    while w >= 1:
        h = h[:, :w] + h[:, w:]
        w //= 2
    ss_ref[...] = h[:, 0][None, None, :]
    l = el_ref[...]
    ml_ref[...] = jnp.max(l, axis=1)[None, None, :]


def _reduce_pass(eo2, el2, n):
    nb = n // _BL
    return pl.pallas_call(
        _reduce_body,
        grid=(nb,),
        in_specs=[
            pl.BlockSpec((_BL, 256), lambda i: (i, 0)),
            pl.BlockSpec((_BL, 80), lambda i: (i, 0)),
        ],
        out_specs=[
            pl.BlockSpec((1, 1, _BL), lambda i: (i, 0, 0)),
            pl.BlockSpec((1, 1, _BL), lambda i: (i, 0, 0)),
        ],
        out_shape=[
            jax.ShapeDtypeStruct((nb, 1, _BL), jnp.float32),
            jax.ShapeDtypeStruct((nb, 1, _BL), jnp.float32),
        ],
    )(eo2, el2)


def _score_body(en_ref, cp_ref, out_ref):
    e = en_ref[0]
    zr = jnp.zeros((1, 256), jnp.float32)
    zc = jnp.zeros((256, 1), jnp.float32)

    def sh(a, dh, dw):
        if dh == 1:
            a = jnp.concatenate([a[1:, :], zr], axis=0)
        elif dh == -1:
            a = jnp.concatenate([zr, a[:-1, :]], axis=0)
        if dw == 1:
            a = jnp.concatenate([a[:, 1:], zc], axis=1)
        elif dw == -1:
            a = jnp.concatenate([zc, a[:, :-1]], axis=1)
        return a

    win = sh(e, -1, -1)
    for dh, dw in ((-1, 0), (-1, 1), (0, -1), (0, 0), (0, 1),
                   (1, -1), (1, 0), (1, 1)):
        win = win + sh(e, dh, dw)
    dens = win * jnp.float32(1.0 / 9.0)
    mn = jnp.min(dens)
    mx = jnp.max(dens)
    denom = (mx - mn) + 1e-06
    cp = cp_ref[0]
    out_ref[0] = cp * (1.0 - 0.4) + ((dens - mn) / denom) * 0.4


def _score_pass(energy, cp):
    return pl.pallas_call(
        _score_body,
        grid=(2,),
        in_specs=[
            pl.BlockSpec((1, 256, 256), lambda b: (b, 0, 0)),
            pl.BlockSpec((1, 256, 256), lambda b: (b, 0, 0)),
        ],
        out_specs=pl.BlockSpec((1, 256, 256), lambda b: (b, 0, 0)),
        out_shape=jax.ShapeDtypeStruct((2, 256, 256), jnp.float32),
    )(energy, cp)


def kernel(enc_outputs, enc_logits):
    B, L, C = enc_outputs.shape
    eo2 = enc_outputs.reshape(B * L, C)
    el2 = enc_logits.reshape(B * L, enc_logits.shape[-1])
    ss, ml = _reduce_pass(eo2, el2, B * L)
    energy = jnp.sqrt(ss.reshape(B, 256, 256))
    cp = jax.nn.sigmoid(ml.reshape(B, 256, 256))
    score = _score_pass(energy, cp).reshape(B, L)
    topk_scores, topk_indexes = jax.lax.top_k(score, 300)
    return (topk_indexes, topk_scores)


# speedup vs baseline: 1.9293x; 1.9293x over previous
import jax
import jax.numpy as jnp
from jax.experimental import pallas as pl

_BL = 2048
_K = 304


def _reduce_body(eo_ref, el_ref, en_ref, cp_ref):
    x = eo_ref[...]
    ss = jnp.sum(x * x, axis=1, keepdims=True)
    en_ref[0] = jnp.sqrt(ss).reshape(_BL // 256, 256)
    ml = jnp.max(el_ref[...], axis=1, keepdims=True)
    cp_ref[0] = jax.nn.sigmoid(ml).reshape(_BL // 256, 256)


def _reduce_pass(eo2, el2, B, L):
    n = B * L
    nb = n // _BL
    rows = _BL // 256
    bpb = L // _BL
    return pl.pallas_call(
        _reduce_body,
        grid=(nb,),
        in_specs=[
            pl.BlockSpec((_BL, 256), lambda i: (i, 0)),
            pl.BlockSpec((_BL, 80), lambda i: (i, 0)),
        ],
        out_specs=[
            pl.BlockSpec((1, rows, 256), lambda i: (i // bpb, i % bpb, 0)),
            pl.BlockSpec((1, rows, 256), lambda i: (i // bpb, i % bpb, 0)),
        ],
        out_shape=[
            jax.ShapeDtypeStruct((B, 256, 256), jnp.float32),
            jax.ShapeDtypeStruct((B, 256, 256), jnp.float32),
        ],
    )(eo2, el2)


def _score_one(e, cp):
    zr = jnp.zeros((1, 256), jnp.float32)
    zc = jnp.zeros((256, 1), jnp.float32)

    def sh(a, dh, dw):
        if dh == 1:
            a = jnp.concatenate([a[1:, :], zr], axis=0)
        elif dh == -1:
            a = jnp.concatenate([zr, a[:-1, :]], axis=0)
        if dw == 1:
            a = jnp.concatenate([a[:, 1:], zc], axis=1)
        elif dw == -1:
            a = jnp.concatenate([zc, a[:, :-1]], axis=1)
        return a

    win = sh(e, -1, -1)
    for dh, dw in ((-1, 0), (-1, 1), (0, -1), (0, 0), (0, 1),
                   (1, -1), (1, 0), (1, 1)):
        win = win + sh(e, dh, dw)
    dens = win * jnp.float32(1.0 / 9.0)
    mn = jnp.min(dens)
    mx = jnp.max(dens)
    denom = (mx - mn) + 1e-06
    return cp * (1.0 - 0.4) + ((dens - mn) / denom) * 0.4


def _score_topk_body(en_ref, cp_ref, val_ref, idx_ref):
    r = jax.lax.broadcasted_iota(jnp.int32, (256, 256), 0)
    c = jax.lax.broadcasted_iota(jnp.int32, (256, 256), 1)
    lin = r * 256 + c
    big = jnp.int32(1 << 30)

    s0 = _score_one(en_ref[0], cp_ref[0])
    s1 = _score_one(en_ref[1], cp_ref[1])
    b0 = jax.lax.bitcast_convert_type(s0, jnp.int32)
    b1 = jax.lax.bitcast_convert_type(s1, jnp.int32)

    def body(t, carry):
        b0, b1 = carry
        m0 = jnp.max(b0)
        m1 = jnp.max(b1)
        i0 = jnp.min(jnp.where(b0 == m0, lin, big))
        i1 = jnp.min(jnp.where(b1 == m1, lin, big))
        val_ref[0, pl.ds(t, 1), 0] = jax.lax.bitcast_convert_type(
            m0, jnp.float32)[None]
        val_ref[1, pl.ds(t, 1), 0] = jax.lax.bitcast_convert_type(
            m1, jnp.float32)[None]
        idx_ref[0, pl.ds(t, 1), 0] = i0[None]
        idx_ref[1, pl.ds(t, 1), 0] = i1[None]
        return (jnp.where(lin == i0, jnp.int32(-1), b0),
                jnp.where(lin == i1, jnp.int32(-1), b1))

    jax.lax.fori_loop(0, 300, body, (b0, b1))


def _score_topk_pass(energy, cp):
    return pl.pallas_call(
        _score_topk_body,
        grid=(1,),
        in_specs=[
            pl.BlockSpec((2, 256, 256), lambda i: (0, 0, 0)),
            pl.BlockSpec((2, 256, 256), lambda i: (0, 0, 0)),
        ],
        out_specs=[
            pl.BlockSpec((2, _K, 1), lambda i: (0, 0, 0)),
            pl.BlockSpec((2, _K, 1), lambda i: (0, 0, 0)),
        ],
        out_shape=[
            jax.ShapeDtypeStruct((2, _K, 1), jnp.float32),
            jax.ShapeDtypeStruct((2, _K, 1), jnp.int32),
        ],
    )(energy, cp)


def kernel(enc_outputs, enc_logits):
    B, L, C = enc_outputs.shape
    eo2 = enc_outputs.reshape(B * L, C)
    el2 = enc_logits.reshape(B * L, enc_logits.shape[-1])
    energy, cp = _reduce_pass(eo2, el2, B, L)
    vals, idxs = _score_topk_pass(energy, cp)
    return (idxs[:, :300, 0], vals[:, :300, 0])
